# baseline (speedup 1.0000x reference)
import jax
import jax.numpy as jnp
from jax import lax
from jax.experimental import pallas as pl
from jax.experimental.pallas import tpu as pltpu
from jax.experimental.pallas import tpu_sc as plsc

VOCAB = 100000
SEQ = 2048
DIM = 128
BATCH = 16

NC = 2
NS = 16
NW = NC * NS
LANES = 16
VPR = DIM // LANES

PW = SEQ // NW
BPS = 2
STEPS = BATCH // BPS
RPS = BPS * PW
NBUF = 5


def _add_pos(rows_v, pos_v, k):
    @plsc.parallel_loop(0, RPS, step=1, unroll=4)
    def _(r):
        prow = lax.rem(r, PW)
        for u in range(VPR):
            off = u * LANES
            x = pos_v[prow, pl.ds(off, LANES)]
            plsc.addupdate(rows_v.at[k, r, pl.ds(off, LANES)], x)


def _body(vals_hbm, tok_hbm, pos_hbm, out_hbm, idx_v, pos_v, rows_v,
          gsem, ssem):
    cid = lax.axis_index("c")
    sid = lax.axis_index("s")
    wid = sid * NC + cid
    p0 = wid * PW

    idx_cps = [
        pltpu.async_copy(vals_hbm.at[pl.ds(b * SEQ + p0, PW)],
                         idx_v.at[b], gsem)
        for b in range(BATCH)
    ]
    pltpu.sync_copy(pos_hbm.at[pl.ds(p0, PW)], pos_v)
    for cp in idx_cps:
        cp.wait()

    gathers = [None] * STEPS
    stores = [None] * STEPS

    def start_gather(s):
        k = s % NBUF
        cps = []
        for j in range(BPS):
            b = s * BPS + j
            cps.append(pltpu.async_copy(
                tok_hbm.at[idx_v.at[b]],
                rows_v.at[k, pl.ds(j * PW, PW)], gsem))
        gathers[s] = cps

    def start_store(s):
        k = s % NBUF
        cps = []
        for j in range(BPS):
            b = s * BPS + j
            cps.append(pltpu.async_copy(
                rows_v.at[k, pl.ds(j * PW, PW)],
                out_hbm.at[pl.ds(b * SEQ + p0, PW)], ssem))
        stores[s] = cps

    for s in range(NBUF - 2):
        start_gather(s)

    for s in range(STEPS):
        k = s % NBUF
        for cp in gathers[s]:
            cp.wait()
        _add_pos(rows_v, pos_v, k)
        start_store(s)
        ns = s + NBUF - 2
        if ns < STEPS:
            if s >= 2:
                for cp in stores[s - 2]:
                    cp.wait()
            start_gather(ns)

    for s in range(max(0, STEPS - NBUF), STEPS):
        for cp in stores[s]:
            cp.wait()


@jax.jit
def kernel(values, tok_table, pos_table):
    vals = values.reshape(BATCH * SEQ).astype(jnp.int32)
    mesh = plsc.VectorSubcoreMesh(core_axis_name="c", subcore_axis_name="s")
    out = pl.kernel(
        _body,
        out_type=jax.ShapeDtypeStruct((BATCH * SEQ, DIM), jnp.float32),
        mesh=mesh,
        scratch_types=[
            pltpu.VMEM((BATCH, PW), jnp.int32),
            pltpu.VMEM((PW, DIM), jnp.float32),
            pltpu.VMEM((NBUF, RPS, DIM), jnp.float32),
            pltpu.SemaphoreType.DMA,
            pltpu.SemaphoreType.DMA,
        ],
    )(vals, tok_table, pos_table)
    return out.reshape(BATCH, SEQ, DIM)


# speedup vs baseline: 1.0173x; 1.0173x over previous
import jax
import jax.numpy as jnp
from jax import lax
from jax.experimental import pallas as pl
from jax.experimental.pallas import tpu as pltpu
from jax.experimental.pallas import tpu_sc as plsc

VOCAB = 100000
SEQ = 2048
DIM = 128
BATCH = 16

NC = 2
NS = 16
NW = NC * NS
LANES = 16
VPR = DIM // LANES

PW = SEQ // NW
BPS = 2
STEPS = BATCH // BPS
RPS = BPS * PW
NBUF = 6
PRIME = 4
AGE = NBUF - PRIME


def _add_pos(rows_v, pos_v, k):
    @plsc.parallel_loop(0, RPS, step=1, unroll=4)
    def _(r):
        prow = lax.rem(r, PW)
        for u in range(VPR):
            off = u * LANES
            x = pos_v[prow, pl.ds(off, LANES)]
            plsc.addupdate(rows_v.at[k, r, pl.ds(off, LANES)], x)


def _body(vals_hbm, tok_hbm, pos_hbm, out_hbm, idx_v, pos_v, rows_v,
          gsem, ssem):
    cid = lax.axis_index("c")
    sid = lax.axis_index("s")
    wid = sid * NC + cid
    p0 = wid * PW

    idx_cps = [
        pltpu.async_copy(vals_hbm.at[pl.ds(b * SEQ + p0, PW)],
                         idx_v.at[b], gsem)
        for b in range(BATCH)
    ]
    pltpu.sync_copy(pos_hbm.at[pl.ds(p0, PW)], pos_v)
    for cp in idx_cps:
        cp.wait()

    gathers = [None] * STEPS
    stores = [None] * STEPS

    def start_gather(s):
        k = s % NBUF
        cps = []
        for j in range(BPS):
            b = s * BPS + j
            cps.append(pltpu.async_copy(
                tok_hbm.at[idx_v.at[b]],
                rows_v.at[k, pl.ds(j * PW, PW)], gsem))
        gathers[s] = cps

    def start_store(s):
        k = s % NBUF
        cps = []
        for j in range(BPS):
            b = s * BPS + j
            cps.append(pltpu.async_copy(
                rows_v.at[k, pl.ds(j * PW, PW)],
                out_hbm.at[pl.ds(b * SEQ + p0, PW)], ssem))
        stores[s] = cps

    for s in range(PRIME):
        start_gather(s)

    for s in range(STEPS):
        k = s % NBUF
        for cp in gathers[s]:
            cp.wait()
        _add_pos(rows_v, pos_v, k)
        start_store(s)
        ns = s + PRIME
        if ns < STEPS:
            if s >= AGE:
                for cp in stores[s - AGE]:
                    cp.wait()
            start_gather(ns)

    for s in range(max(0, STEPS - NBUF), STEPS):
        for cp in stores[s]:
            cp.wait()


@jax.jit
def kernel(values, tok_table, pos_table):
    vals = values.reshape(BATCH * SEQ).astype(jnp.int32)
    mesh = plsc.VectorSubcoreMesh(core_axis_name="c", subcore_axis_name="s")
    out = pl.kernel(
        _body,
        out_type=jax.ShapeDtypeStruct((BATCH * SEQ, DIM), jnp.float32),
        mesh=mesh,
        scratch_types=[
            pltpu.VMEM((BATCH, PW), jnp.int32),
            pltpu.VMEM((PW, DIM), jnp.float32),
            pltpu.VMEM((NBUF, RPS, DIM), jnp.float32),
            pltpu.SemaphoreType.DMA,
            pltpu.SemaphoreType.DMA,
        ],
    )(vals, tok_table, pos_table)
    return out.reshape(BATCH, SEQ, DIM)


# speedup vs baseline: 1.0287x; 1.0112x over previous
import jax
import jax.numpy as jnp
from jax import lax
from jax.experimental import pallas as pl
from jax.experimental.pallas import tpu as pltpu
from jax.experimental.pallas import tpu_sc as plsc

VOCAB = 100000
SEQ = 2048
DIM = 128
BATCH = 16

NC = 2
NS = 16
NW = NC * NS
LANES = 16
VPR = DIM // LANES

PW = SEQ // NW
BPS = 2
STEPS = BATCH // BPS
RPS = BPS * PW
NBUF = 6
PRIME = 4
AGE = NBUF - PRIME


def _add_pos(rows_v, pos_v, k):
    @plsc.parallel_loop(0, RPS, step=1, unroll=4)
    def _(r):
        prow = lax.rem(r, PW)
        for u in range(VPR):
            off = u * LANES
            x = pos_v[prow, pl.ds(off, LANES)]
            plsc.addupdate(rows_v.at[k, r, pl.ds(off, LANES)], x)


def _body(vals_hbm, tok_hbm, pos_hbm, out_hbm, idx_v, pos_v, rows_v,
          gsem, ssem):
    cid = lax.axis_index("c")
    sid = lax.axis_index("s")
    wid = sid * NC + cid
    p0 = wid * PW

    p0_al = pl.multiple_of((wid // 2) * (2 * PW), 2 * PW)
    col = lax.rem(p0, 2 * PW)
    icp = pltpu.async_copy(vals_hbm.at[:, pl.ds(p0_al, 2 * PW)], idx_v, gsem)
    pltpu.sync_copy(pos_hbm.at[pl.ds(p0, PW)], pos_v)
    icp.wait()

    gathers = [None] * STEPS
    stores = [None] * STEPS

    def start_gather(s):
        k = s % NBUF
        cps = []
        for j in range(BPS):
            b = s * BPS + j
            cps.append(pltpu.async_copy(
                tok_hbm.at[idx_v.at[b, pl.ds(col, PW)]],
                rows_v.at[k, pl.ds(j * PW, PW)], gsem))
        gathers[s] = cps

    def start_store(s):
        k = s % NBUF
        cps = []
        for j in range(BPS):
            b = s * BPS + j
            cps.append(pltpu.async_copy(
                rows_v.at[k, pl.ds(j * PW, PW)],
                out_hbm.at[pl.ds(b * SEQ + p0, PW)], ssem))
        stores[s] = cps

    for s in range(PRIME):
        start_gather(s)

    for s in range(STEPS):
        k = s % NBUF
        for cp in gathers[s]:
            cp.wait()
        _add_pos(rows_v, pos_v, k)
        start_store(s)
        ns = s + PRIME
        if ns < STEPS:
            if s >= AGE:
                for cp in stores[s - AGE]:
                    cp.wait()
            start_gather(ns)

    for s in range(max(0, STEPS - NBUF), STEPS):
        for cp in stores[s]:
            cp.wait()


@jax.jit
def kernel(values, tok_table, pos_table):
    vals = values.astype(jnp.int32)
    mesh = plsc.VectorSubcoreMesh(core_axis_name="c", subcore_axis_name="s")
    out = pl.kernel(
        _body,
        out_type=jax.ShapeDtypeStruct((BATCH * SEQ, DIM), jnp.float32),
        mesh=mesh,
        scratch_types=[
            pltpu.VMEM((BATCH, 2 * PW), jnp.int32),
            pltpu.VMEM((PW, DIM), jnp.float32),
            pltpu.VMEM((NBUF, RPS, DIM), jnp.float32),
            pltpu.SemaphoreType.DMA,
            pltpu.SemaphoreType.DMA,
        ],
    )(vals, tok_table, pos_table)
    return out.reshape(BATCH, SEQ, DIM)


# speedup vs baseline: 1.0431x; 1.0140x over previous
import jax
import jax.numpy as jnp
from jax import lax
from jax.experimental import pallas as pl
from jax.experimental.pallas import tpu as pltpu
from jax.experimental.pallas import tpu_sc as plsc

VOCAB = 100000
SEQ = 2048
DIM = 128
BATCH = 16

NC = 2
NS = 16
NW = NC * NS
LANES = 16
VPR = DIM // LANES

PW = SEQ // NW
BPS = 2
STEPS = BATCH // BPS
RPS = BPS * PW
NBUF = 4
PRIME = 3
OUTER = STEPS // NBUF


def _add_pos(rows_v, pos_v, k):
    @plsc.parallel_loop(0, RPS, step=1, unroll=4)
    def _(r):
        prow = lax.rem(r, PW)
        for u in range(VPR):
            off = u * LANES
            x = pos_v[prow, pl.ds(off, LANES)]
            plsc.addupdate(rows_v.at[k, r, pl.ds(off, LANES)], x)


def _body(vals_hbm, tok_hbm, pos_hbm, out_hbm, idx_v, pos_v, rows_v,
          gsem, ssem):
    cid = lax.axis_index("c")
    sid = lax.axis_index("s")
    wid = sid * NC + cid
    p0 = wid * PW

    p0_al = pl.multiple_of((wid // 2) * (2 * PW), 2 * PW)
    col = lax.rem(p0, 2 * PW)
    icp = pltpu.async_copy(vals_hbm.at[:, pl.ds(p0_al, 2 * PW)], idx_v, gsem)
    pltpu.sync_copy(pos_hbm.at[pl.ds(p0, PW)], pos_v)
    icp.wait()

    def start_gather(s, k):
        for j in range(BPS):
            b = s * BPS + j
            pltpu.async_copy(
                tok_hbm.at[idx_v.at[b, pl.ds(col, PW)]],
                rows_v.at[k, pl.ds(j * PW, PW)], gsem)

    def start_store(s, k):
        for j in range(BPS):
            b = s * BPS + j
            pltpu.async_copy(
                rows_v.at[k, pl.ds(j * PW, PW)],
                out_hbm.at[pl.ds(b * SEQ + p0, PW)], ssem)

    def wait_gather(k):
        for j in range(BPS):
            pltpu.make_async_copy(
                tok_hbm.at[pl.ds(0, PW)],
                rows_v.at[k, pl.ds(j * PW, PW)], gsem).wait()

    def wait_store(k):
        for j in range(BPS):
            pltpu.make_async_copy(
                rows_v.at[k, pl.ds(j * PW, PW)],
                out_hbm.at[pl.ds(0, PW)], ssem).wait()

    for s in range(PRIME):
        start_gather(s, s % NBUF)

    def outer(o, _):
        s0 = o * NBUF
        for i in range(NBUF):
            s = s0 + i
            wait_gather(i)
            _add_pos(rows_v, pos_v, i)
            start_store(s, i)
            ns = s + PRIME

            @pl.when(ns < STEPS)
            def _():
                @pl.when(s >= 1)
                def _():
                    wait_store((i - 1) % NBUF)

                start_gather(ns, (i + PRIME) % NBUF)
        return 0

    lax.fori_loop(0, OUTER, outer, 0)

    for i in range(NBUF):
        wait_store(i)


@jax.jit
def kernel(values, tok_table, pos_table):
    vals = values.astype(jnp.int32)
    mesh = plsc.VectorSubcoreMesh(core_axis_name="c", subcore_axis_name="s")
    out = pl.kernel(
        _body,
        out_type=jax.ShapeDtypeStruct((BATCH * SEQ, DIM), jnp.float32),
        mesh=mesh,
        scratch_types=[
            pltpu.VMEM((BATCH, 2 * PW), jnp.int32),
            pltpu.VMEM((PW, DIM), jnp.float32),
            pltpu.VMEM((NBUF, RPS, DIM), jnp.float32),
            pltpu.SemaphoreType.DMA,
            pltpu.SemaphoreType.DMA,
        ],
    )(vals, tok_table, pos_table)
    return out.reshape(BATCH, SEQ, DIM)
